# baseline (speedup 1.0000x reference)
import jax
import jax.numpy as jnp
from jax.experimental import pallas as pl
from jax.experimental.pallas import tpu as pltpu

_NB = 4


def _merge3(a, b, lo_of, hi_of):
    a1, a2, a3 = a
    b1, b2, b3 = b
    c1 = hi_of(a1, b1)
    c2 = hi_of(hi_of(a2, b2), lo_of(a1, b1))
    c3 = hi_of(hi_of(a3, b3), hi_of(lo_of(a2, b1), lo_of(a1, b2)))
    return c1, c2, c3


def _select3(f):
    mx = jnp.maximum
    mn = jnp.minimum
    n = f.shape[0]

    half = n // 2
    hi = mx(f[:half], f[half:])
    lo = mn(f[:half], f[half:])

    q = half // 2
    a1, b1 = hi[:q], hi[q:]
    a2, b2 = lo[:q], lo[q:]
    p = mn(a1, b1)
    r = mx(a2, b2)
    s1 = mx(a1, b1)
    s2 = mx(p, r)
    s3 = mn(p, r)
    s4 = mn(a2, b2)

    e = q // 2
    A1, B1 = s1[:e], s1[e:]
    A2, B2 = s2[:e], s2[e:]
    A3, B3 = s3[:e], s3[e:]
    A4, B4 = s4[:e], s4[e:]
    top = (mx(A1, B1),
           mx(mn(A1, B1), mx(A2, B2)),
           mx(mx(A3, B3), mx(mn(A2, B1), mn(A1, B2))))
    bot = (mn(A4, B4),
           mn(mx(A4, B4), mn(A3, B3)),
           mn(mn(A2, B2), mn(mx(A3, B4), mx(A4, B3))))

    rows = e
    while rows > 1:
        h = rows // 2
        top = _merge3(tuple(t[:h] for t in top),
                      tuple(t[h:] for t in top), mn, mx)
        bot = _merge3(tuple(t[:h] for t in bot),
                      tuple(t[h:] for t in bot), mx, mn)
        rows = h
    return top[2], bot[2]


def _weldon_body(x_ref, w_ref, out_ref):
    w = w_ref[...]
    zero = jnp.float32(0.0)
    for b in range(_NB):
        f = jnp.dot(x_ref[b], w, preferred_element_type=jnp.float32)
        t3, b3 = _select3(f)
        contrib = jnp.where(f >= t3, f, zero) + jnp.where(f <= b3, f, zero)
        pooled = jnp.sum(contrib, axis=0, keepdims=True)
        sq = jnp.sum(pooled * pooled, axis=1, keepdims=True)
        out_ref[b] = pooled * jax.lax.rsqrt(jnp.maximum(sq, jnp.float32(1e-12)))


def kernel(x, W):
    B, H, Wsp, C = x.shape
    D = W.shape[1]
    N = H * Wsp
    xr = x.reshape(B, N, C)
    return pl.pallas_call(
        _weldon_body,
        grid=(B // _NB,),
        in_specs=[
            pl.BlockSpec((_NB, N, C), lambda b: (b, 0, 0)),
            pl.BlockSpec((C, D), lambda b: (0, 0)),
        ],
        out_specs=pl.BlockSpec((_NB, 1, D), lambda b: (b, 0, 0)),
        out_shape=jax.ShapeDtypeStruct((B, 1, D), jnp.float32),
    )(xr, W).reshape(B, D)


# speedup vs baseline: 1.2422x; 1.2422x over previous
import jax
import jax.numpy as jnp
from jax.experimental import pallas as pl
from jax.experimental.pallas import tpu as pltpu

_NB = 8


def _merge3(a, b, lo_of, hi_of):
    a1, a2, a3 = a
    b1, b2, b3 = b
    c1 = hi_of(a1, b1)
    c2 = hi_of(hi_of(a2, b2), lo_of(a1, b1))
    c3 = hi_of(hi_of(a3, b3), hi_of(lo_of(a2, b1), lo_of(a1, b2)))
    return c1, c2, c3


def _select3_from_pairs(hi, lo):
    mx = jnp.maximum
    mn = jnp.minimum
    half = hi.shape[0]

    q = half // 2
    a1, b1 = hi[:q], hi[q:]
    a2, b2 = lo[:q], lo[q:]
    p = mn(a1, b1)
    r = mx(a2, b2)
    s1 = mx(a1, b1)
    s2 = mx(p, r)
    s3 = mn(p, r)
    s4 = mn(a2, b2)

    e = q // 2
    A1, B1 = s1[:e], s1[e:]
    A2, B2 = s2[:e], s2[e:]
    A3, B3 = s3[:e], s3[e:]
    A4, B4 = s4[:e], s4[e:]
    top = (mx(A1, B1),
           mx(mn(A1, B1), mx(A2, B2)),
           mx(mx(A3, B3), mx(mn(A2, B1), mn(A1, B2))))
    bot = (mn(A4, B4),
           mn(mx(A4, B4), mn(A3, B3)),
           mn(mn(A2, B2), mn(mx(A3, B4), mx(A4, B3))))

    rows = e
    while rows > 1:
        h = rows // 2
        top = _merge3(tuple(t[:h] for t in top),
                      tuple(t[h:] for t in top), mn, mx)
        bot = _merge3(tuple(t[:h] for t in bot),
                      tuple(t[h:] for t in bot), mx, mn)
        rows = h
    return top[2], bot[2]


def _weldon_body(x_ref, w_ref, out_ref):
    w = w_ref[...]
    zero = jnp.float32(0.0)
    for b in range(_NB):
        f = jnp.dot(x_ref[b], w, preferred_element_type=jnp.float32)
        f4 = f.reshape(64, 16, 128)
        hi = jnp.maximum(f4[:, :8, :], f4[:, 8:, :]).reshape(512, 128)
        lo = jnp.minimum(f4[:, :8, :], f4[:, 8:, :]).reshape(512, 128)
        t3, b3 = _select3_from_pairs(hi, lo)
        contrib = (jnp.where(hi >= t3, hi, zero) + jnp.where(hi <= b3, hi, zero)
                   + jnp.where(lo >= t3, lo, zero) + jnp.where(lo <= b3, lo, zero))
        pooled = jnp.sum(contrib, axis=0, keepdims=True)
        sq = jnp.sum(pooled * pooled, axis=1, keepdims=True)
        out_ref[b] = pooled * jax.lax.rsqrt(jnp.maximum(sq, jnp.float32(1e-12)))


def kernel(x, W):
    B, H, Wsp, C = x.shape
    D = W.shape[1]
    N = H * Wsp
    xr = x.reshape(B, N, C)
    return pl.pallas_call(
        _weldon_body,
        grid=(B // _NB,),
        in_specs=[
            pl.BlockSpec((_NB, N, C), lambda b: (b, 0, 0)),
            pl.BlockSpec((C, D), lambda b: (0, 0)),
        ],
        out_specs=pl.BlockSpec((_NB, 1, D), lambda b: (b, 0, 0)),
        out_shape=jax.ShapeDtypeStruct((B, 1, D), jnp.float32),
    )(xr, W).reshape(B, D)
